# Initial kernel scaffold; baseline (speedup 1.0000x reference)
import jax
import jax.numpy as jnp
from jax.experimental import pallas as pl


def kernel(input, params):
    raise NotImplementedError("write your pallas kernel here")


# speedup vs baseline: 67.5184x; 67.5184x over previous
import functools

import numpy as np
import jax
import jax.numpy as jnp
from jax.experimental import pallas as pl

_K = 16
_DEC = 2
_SPECS = [(8, 16), (32, 64), (128, 128), (256, 256)]
_F32 = jnp.float32


def _leaky(x, slope):
    return jnp.where(x >= 0, x, slope * x)


def _dot(a, b):
    return jnp.dot(a, b, preferred_element_type=_F32)


def _attpool(s, x, tq, d):
    s_parts = [s[k * tq:(k + 1) * tq, :] for k in range(_K)]
    x_parts = [x[k * tq:(k + 1) * tq, :] for k in range(_K)]
    m = functools.reduce(jnp.maximum, s_parts)
    e_parts = [jnp.exp(p - m) for p in s_parts]
    z = functools.reduce(jnp.add, e_parts)
    inv_z = 1.0 / z
    acc = None
    for e, xp in zip(e_parts, x_parts):
        t = (e * inv_z) * xp
        acc = t if acc is None else acc + t
    return acc


def _make_layer_body(tq, cur, d_in, d_out, first, last):
    h = d_out // 2

    def body(qc_ref, cand_ref, candt_ref, feats_ref, *refs):
        ws = [r[...] for r in refs[:-1]]
        out_ref = refs[-1]
        it = iter(ws)
        if first:
            w_fc, b_fc = next(it), next(it)
        w_mlp1, b_mlp1 = next(it), next(it)
        w_lse1, b_lse1 = next(it), next(it)
        w_score1 = next(it)
        w_pool1, b_pool1 = next(it), next(it)
        w_lse2, b_lse2 = next(it), next(it)
        w_score2 = next(it)
        w_pool2, b_pool2 = next(it), next(it)
        w_mlp2, b_mlp2 = next(it), next(it)
        w_sc, b_sc = next(it), next(it)
        if last:
            w_final, b_final = next(it), next(it)
            w_proj, b_proj = next(it), next(it)

        q8 = qc_ref[0]
        c8 = cand_ref[0]
        ct = candt_ref[0]
        f_raw = feats_ref[0]

        if first:
            f_in = _dot(f_raw, w_fc) + b_fc
        else:
            f_in = f_raw

        sq_q = jnp.sum(q8 * q8, axis=1, keepdims=True)
        sq_c = jnp.sum(ct * ct, axis=0, keepdims=True)
        d2 = sq_q + sq_c - 2.0 * _dot(q8, ct)
        d2 = jnp.maximum(d2, 0.0)

        iota = jax.lax.broadcasted_iota(jnp.int32, (tq, cur), 1)
        q3 = q8[:, :3]
        cats = []
        for _ in range(_K):
            m = jnp.min(d2, axis=1, keepdims=True)
            idxs = jnp.where(d2 == m, iota, cur)
            amin = jnp.min(idxs, axis=1, keepdims=True)
            onehot = iota == amin
            nb = _dot(onehot.astype(_F32), c8)
            nb3 = nb[:, :3]
            cats.append(jnp.concatenate([q3, nb3, q3 - nb3, m], axis=1))
            d2 = jnp.where(onehot, jnp.array(np.inf, _F32), d2)
        cat = jnp.concatenate(cats, axis=0)

        f1 = _leaky(_dot(f_in, w_mlp1) + b_mlp1, 0.2)
        enc1 = jnp.maximum(_dot(cat, w_lse1) + b_lse1, 0.0)
        f1t = jnp.concatenate([f1] * _K, axis=0)
        x1 = jnp.concatenate([enc1, f1t], axis=1)
        pooled1 = _attpool(_dot(x1, w_score1), x1, tq, d_out)
        f2 = jnp.maximum(_dot(pooled1, w_pool1) + b_pool1, 0.0)

        enc2 = jnp.maximum(_dot(cat, w_lse2) + b_lse2, 0.0)
        f2t = jnp.concatenate([f2] * _K, axis=0)
        x2 = jnp.concatenate([enc2, f2t], axis=1)
        pooled2 = _attpool(_dot(x2, w_score2), x2, tq, d_out)
        f3 = jnp.maximum(_dot(pooled2, w_pool2) + b_pool2, 0.0)

        y = _leaky(_dot(f3, w_mlp2) + b_mlp2 + _dot(f_in, w_sc) + b_sc, 0.01)

        if last:
            z = jnp.maximum(_dot(y, w_final) + b_final, 0.0)
            y = _dot(z, w_proj) + b_proj

        out_ref[0] = y

    return body


def _wt(w):
    return jnp.transpose(w, (1, 0))


def _b2(b):
    return b[None, :]


def _layer_call(i, feats, qc, cand, candt, lp, extra_first, extra_last, tq):
    first = i == 0
    last = i == len(_SPECS) - 1
    d_in, d_out = _SPECS[i]
    B, Q, d_feat = feats.shape
    cur = cand.shape[1]

    wlist = []
    if first:
        wlist += list(extra_first)
    wlist += [
        _wt(lp['w_mlp1']), _b2(lp['b_mlp1']),
        _wt(lp['w_lse1']), _b2(lp['b_lse1']),
        _wt(lp['w_score1']),
        _wt(lp['w_pool1']), _b2(lp['b_pool1']),
        _wt(lp['w_lse2']), _b2(lp['b_lse2']),
        _wt(lp['w_score2']),
        _wt(lp['w_pool2']), _b2(lp['b_pool2']),
        _wt(lp['w_mlp2']), _b2(lp['b_mlp2']),
        _wt(lp['w_sc']), _b2(lp['b_sc']),
    ]
    if last:
        wlist += list(extra_last)

    d_res = wlist[-1].shape[-1] if last else 2 * d_out

    grid = (B, Q // tq)
    w_specs = [
        pl.BlockSpec(w.shape, lambda b, t, nd=w.ndim: (0,) * nd) for w in wlist
    ]
    in_specs = [
        pl.BlockSpec((1, tq, 8), lambda b, t: (b, t, 0)),
        pl.BlockSpec((1, cur, 8), lambda b, t: (b, 0, 0)),
        pl.BlockSpec((1, 8, cur), lambda b, t: (b, 0, 0)),
        pl.BlockSpec((1, tq, d_feat), lambda b, t: (b, t, 0)),
    ] + w_specs

    body = _make_layer_body(tq, cur, d_in, d_out, first, last)
    return pl.pallas_call(
        body,
        grid=grid,
        in_specs=in_specs,
        out_specs=pl.BlockSpec((1, tq, d_res), lambda b, t: (b, t, 0)),
        out_shape=jax.ShapeDtypeStruct((B, Q, d_res), _F32),
    )(qc, cand, candt, feats, *wlist)


def kernel(input, params):
    B, N, _ = input.shape
    Q = N // (_DEC ** len(_SPECS))
    tq = min(128, Q)

    perm = np.random.RandomState(0).permutation(N)
    xp = input[:, perm, :]
    coords = xp[..., :3]
    cpad = jnp.concatenate(
        [coords, jnp.zeros((B, N, 5), _F32)], axis=-1)
    ct = jnp.transpose(cpad, (0, 2, 1))
    qc = cpad[:, :Q, :]
    finp = jnp.concatenate(
        [xp[:, :Q, :], jnp.zeros((B, Q, 2), _F32)], axis=-1)

    w_fc = jnp.concatenate(
        [_wt(params['w_fc']), jnp.zeros((2, params['w_fc'].shape[0]), _F32)],
        axis=0)
    extra_first = (w_fc, _b2(params['b_fc']))
    extra_last = (
        _wt(params['w_final']), _b2(params['b_final']),
        _wt(params['w_proj']), _b2(params['b_proj']),
    )

    feats = finp
    for i in range(len(_SPECS)):
        cur = N >> i
        feats = _layer_call(
            i, feats, qc, cpad[:, :cur, :], ct[:, :, :cur],
            params['layers'][i], extra_first, extra_last, tq)
    return feats


# speedup vs baseline: 76.3809x; 1.1313x over previous
import functools

import numpy as np
import jax
import jax.numpy as jnp
from jax.experimental import pallas as pl

_K = 16
_DEC = 2
_SPECS = [(8, 16), (32, 64), (128, 128), (256, 256)]
_F32 = jnp.float32


def _leaky(x, slope):
    return jnp.where(x >= 0, x, slope * x)


def _dot(a, b):
    return jnp.dot(a, b, preferred_element_type=_F32)


def _attpool(s, x, tq):
    s_parts = [s[k * tq:(k + 1) * tq, :] for k in range(_K)]
    x_parts = [x[k * tq:(k + 1) * tq, :] for k in range(_K)]
    m = functools.reduce(jnp.maximum, s_parts)
    e_parts = [jnp.exp(p - m) for p in s_parts]
    z = functools.reduce(jnp.add, e_parts)
    acc = None
    for e, xp in zip(e_parts, x_parts):
        t = e * xp
        acc = t if acc is None else acc + t
    return acc * (1.0 / z)


def _make_body(tq, n):
    num_layers = len(_SPECS)

    def body(qc_ref, cand_ref, candt_ref, finp_ref, *refs):
        out_ref = refs[-1]
        ws = [r[...] for r in refs[:-1]]
        it = iter(ws)

        w_fc, b_fc = next(it), next(it)
        layer_ws = []
        for _ in range(num_layers):
            layer_ws.append([next(it) for _ in range(16)])
        w_final, b_final = next(it), next(it)
        w_proj, b_proj = next(it), next(it)

        q8 = qc_ref[0]
        c8f = cand_ref[0]
        ctf = candt_ref[0]
        f_raw = finp_ref[0]

        f_in = _dot(f_raw, w_fc) + b_fc
        q3 = q8[:, :3]
        sq_q = jnp.sum(q8 * q8, axis=1, keepdims=True)
        sq_cf = jnp.sum(ctf * ctf, axis=0, keepdims=True)

        for li in range(num_layers):
            (w_mlp1, b_mlp1, w_lse1, b_lse1, w_score1, w_pool1, b_pool1,
             w_lse2, b_lse2, w_score2, w_pool2, b_pool2, w_mlp2, b_mlp2,
             w_sc, b_sc) = layer_ws[li]
            cur = n >> li

            d2 = sq_q + sq_cf[:, :cur] - 2.0 * _dot(q8, ctf[:, :cur])
            d2 = jnp.maximum(d2, 0.0)

            iota = jax.lax.broadcasted_iota(jnp.int32, (tq, cur), 1)
            cats = []
            for _ in range(_K):
                m = jnp.min(d2, axis=1, keepdims=True)
                idxs = jnp.where(d2 == m, iota, cur)
                amin = jnp.min(idxs, axis=1, keepdims=True)
                onehot = iota == amin
                nb = _dot(onehot.astype(_F32), c8f[:cur])
                d2 = jnp.where(onehot, jnp.array(np.inf, _F32), d2)
                nb3 = nb[:, :3]
                cats.append(jnp.concatenate([q3, nb3, q3 - nb3, m], axis=1))
            cat = jnp.concatenate(cats, axis=0)

            f1 = _leaky(_dot(f_in, w_mlp1) + b_mlp1, 0.2)
            enc1 = jnp.maximum(_dot(cat, w_lse1) + b_lse1, 0.0)
            x1 = jnp.concatenate([enc1, jnp.concatenate([f1] * _K, axis=0)],
                                 axis=1)
            pooled1 = _attpool(_dot(x1, w_score1), x1, tq)
            f2 = jnp.maximum(_dot(pooled1, w_pool1) + b_pool1, 0.0)

            enc2 = jnp.maximum(_dot(cat, w_lse2) + b_lse2, 0.0)
            x2 = jnp.concatenate([enc2, jnp.concatenate([f2] * _K, axis=0)],
                                 axis=1)
            pooled2 = _attpool(_dot(x2, w_score2), x2, tq)
            f3 = jnp.maximum(_dot(pooled2, w_pool2) + b_pool2, 0.0)

            f_in = _leaky(
                _dot(f3, w_mlp2) + b_mlp2 + _dot(f_in, w_sc) + b_sc, 0.01)

        z = jnp.maximum(_dot(f_in, w_final) + b_final, 0.0)
        out_ref[0] = _dot(z, w_proj) + b_proj

    return body


def _wt(w):
    return jnp.transpose(w, (1, 0))


def _b2(b):
    return b[None, :]


def kernel(input, params):
    B, N, _ = input.shape
    Q = N // (_DEC ** len(_SPECS))
    tq = min(128, Q)

    perm = np.random.RandomState(0).permutation(N)
    xp = input[:, perm, :]
    coords = xp[..., :3]
    cpad = jnp.concatenate(
        [coords, jnp.zeros((B, N, 5), _F32)], axis=-1)
    ct = jnp.transpose(cpad, (0, 2, 1))
    qc = cpad[:, :Q, :]
    finp = jnp.concatenate(
        [xp[:, :Q, :], jnp.zeros((B, Q, 2), _F32)], axis=-1)

    w_fc = jnp.concatenate(
        [_wt(params['w_fc']), jnp.zeros((2, params['w_fc'].shape[0]), _F32)],
        axis=0)
    wlist = [w_fc, _b2(params['b_fc'])]
    for lp in params['layers']:
        wlist += [
            _wt(lp['w_mlp1']), _b2(lp['b_mlp1']),
            _wt(lp['w_lse1']), _b2(lp['b_lse1']),
            _wt(lp['w_score1']),
            _wt(lp['w_pool1']), _b2(lp['b_pool1']),
            _wt(lp['w_lse2']), _b2(lp['b_lse2']),
            _wt(lp['w_score2']),
            _wt(lp['w_pool2']), _b2(lp['b_pool2']),
            _wt(lp['w_mlp2']), _b2(lp['b_mlp2']),
            _wt(lp['w_sc']), _b2(lp['b_sc']),
        ]
    wlist += [
        _wt(params['w_final']), _b2(params['b_final']),
        _wt(params['w_proj']), _b2(params['b_proj']),
    ]
    d_res = params['w_proj'].shape[0]

    w_specs = [
        pl.BlockSpec(w.shape, lambda b, t, nd=w.ndim: (0,) * nd) for w in wlist
    ]
    in_specs = [
        pl.BlockSpec((1, tq, 8), lambda b, t: (b, t, 0)),
        pl.BlockSpec((1, N, 8), lambda b, t: (b, 0, 0)),
        pl.BlockSpec((1, 8, N), lambda b, t: (b, 0, 0)),
        pl.BlockSpec((1, tq, 8), lambda b, t: (b, t, 0)),
    ] + w_specs

    return pl.pallas_call(
        _make_body(tq, N),
        grid=(B, Q // tq),
        in_specs=in_specs,
        out_specs=pl.BlockSpec((1, tq, d_res), lambda b, t: (b, t, 0)),
        out_shape=jax.ShapeDtypeStruct((B, Q, d_res), _F32),
    )(qc, cpad, ct, finp, *wlist)
